# baseline (speedup 1.0000x reference)
import functools

import jax
import jax.numpy as jnp
from jax import lax
from jax.experimental import pallas as pl
from jax.experimental.pallas import tpu as pltpu
from jax.experimental.pallas import tpu_sc as plsc

NUM_CORES = 2
NUM_SUBCORES = 16
LANES = 16
NW = NUM_CORES * NUM_SUBCORES

TROWS = 200
TCOLS = 16384
COLS_PER_WORKER = TCOLS // NW
CHUNK_COLS = 128
N_CHUNKS = COLS_PER_WORKER // CHUNK_COLS
VOCAB_PAD = 1024


def _make_sc_gather(vocab):
    mesh = plsc.VectorSubcoreMesh(
        core_axis_name="c", subcore_axis_name="s", num_cores=NUM_CORES
    )

    @functools.partial(
        pl.kernel,
        mesh=mesh,
        out_type=jax.ShapeDtypeStruct((TROWS, TCOLS), jnp.float32),
        scratch_types=[
            pltpu.VMEM((VOCAB_PAD,), jnp.float32),
            pltpu.VMEM((2, TROWS, CHUNK_COLS), jnp.int32),
            pltpu.VMEM((2, TROWS, CHUNK_COLS), jnp.float32),
            pltpu.SemaphoreType.DMA,
            pltpu.SemaphoreType.DMA,
            pltpu.SemaphoreType.DMA,
            pltpu.SemaphoreType.DMA,
        ],
        compiler_params=pltpu.CompilerParams(
            needs_layout_passes=False, use_tc_tiling_on_sc=True
        ),
    )
    def sc_gather(
        table_hbm, idx_hbm, out_hbm, tab_v, idx_v, val_v,
        sem_i0, sem_i1, sem_o0, sem_o1,
    ):
        sem_i = (sem_i0, sem_i1)
        sem_o = (sem_o0, sem_o1)
        wid = lax.axis_index("s") * NUM_CORES + lax.axis_index("c")
        base = wid * COLS_PER_WORKER

        def in_slice(c):
            c0 = pl.multiple_of(base + c * CHUNK_COLS, CHUNK_COLS)
            return idx_hbm.at[:, pl.ds(c0, CHUNK_COLS)]

        def out_slice(c):
            c0 = pl.multiple_of(base + c * CHUNK_COLS, CHUNK_COLS)
            return out_hbm.at[:, pl.ds(c0, CHUNK_COLS)]

        def start_in(c, b):
            pltpu.async_copy(in_slice(c), idx_v.at[b], sem_i[b])

        def wait_in(b):
            pltpu.make_async_copy(in_slice(0), idx_v.at[b], sem_i[b]).wait()

        def start_out(c, b):
            pltpu.async_copy(val_v.at[b], out_slice(c), sem_o[b])

        def wait_out(b):
            pltpu.make_async_copy(val_v.at[b], out_slice(0), sem_o[b]).wait()

        start_in(0, 0)
        start_in(1, 1)
        pltpu.sync_copy(table_hbm, tab_v.at[pl.ds(0, vocab)])

        def gather_chunk(b):
            idx_b = idx_v.at[b]
            val_b = val_v.at[b]

            def gather_row(r):
                for k in range(CHUNK_COLS // LANES):
                    iv = idx_b[r, pl.ds(k * LANES, LANES)]
                    val_b[r, pl.ds(k * LANES, LANES)] = plsc.load_gather(
                        tab_v, [iv]
                    )

            plsc.parallel_loop(0, TROWS, unroll=1)(gather_row)

        def pair(p, _):
            c = p * 2
            for b in range(2):
                wait_in(b)
                lax.cond(p > 0, lambda: wait_out(b), lambda: None)
                gather_chunk(b)
                start_out(c + b, b)
                lax.cond(
                    c + b + 2 < N_CHUNKS,
                    lambda: start_in(c + b + 2, b),
                    lambda: None,
                )
            return 0

        lax.fori_loop(0, N_CHUNKS // 2, pair, 0)
        wait_out(0)
        wait_out(1)

    return sc_gather


_sc_gather_1000 = _make_sc_gather(1000)


def kernel(index, names_table):
    out_t = _sc_gather_1000(names_table, index.T)
    return out_t.T


# speedup vs baseline: 1.0156x; 1.0156x over previous
import functools

import jax
import jax.numpy as jnp
from jax import lax
from jax.experimental import pallas as pl
from jax.experimental.pallas import tpu as pltpu
from jax.experimental.pallas import tpu_sc as plsc

NUM_CORES = 2
NUM_SUBCORES = 16
LANES = 16
NW = NUM_CORES * NUM_SUBCORES

TROWS = 200
TCOLS = 16384
COLS_PER_WORKER = TCOLS // NW
CHUNK_COLS = 128
N_CHUNKS = COLS_PER_WORKER // CHUNK_COLS
VOCAB_PAD = 1024


def _make_sc_gather(vocab):
    mesh = plsc.VectorSubcoreMesh(
        core_axis_name="c", subcore_axis_name="s", num_cores=NUM_CORES
    )

    @functools.partial(
        pl.kernel,
        mesh=mesh,
        out_type=jax.ShapeDtypeStruct((TROWS, TCOLS), jnp.float32),
        scratch_types=[
            pltpu.VMEM((VOCAB_PAD,), jnp.float32),
            pltpu.VMEM((2, TROWS, CHUNK_COLS), jnp.int32),
            pltpu.VMEM((2, TROWS, CHUNK_COLS), jnp.float32),
            pltpu.SemaphoreType.DMA,
            pltpu.SemaphoreType.DMA,
            pltpu.SemaphoreType.DMA,
            pltpu.SemaphoreType.DMA,
        ],
        compiler_params=pltpu.CompilerParams(
            needs_layout_passes=False, use_tc_tiling_on_sc=True
        ),
    )
    def sc_gather(
        table_hbm, idx_hbm, out_hbm, tab_v, idx_v, val_v,
        sem_i0, sem_i1, sem_o0, sem_o1,
    ):
        sem_i = (sem_i0, sem_i1)
        sem_o = (sem_o0, sem_o1)
        wid = lax.axis_index("s") * NUM_CORES + lax.axis_index("c")
        base = wid * COLS_PER_WORKER

        def start_in(c):
            c0 = base + c * CHUNK_COLS
            return pltpu.async_copy(
                idx_hbm.at[:, pl.ds(c0, CHUNK_COLS)],
                idx_v.at[c % 2],
                sem_i[c % 2],
            )

        def start_out(c):
            c0 = base + c * CHUNK_COLS
            return pltpu.async_copy(
                val_v.at[c % 2],
                out_hbm.at[:, pl.ds(c0, CHUNK_COLS)],
                sem_o[c % 2],
            )

        in_dma = {0: start_in(0)}
        out_dma = {}
        pltpu.sync_copy(table_hbm, tab_v.at[pl.ds(0, vocab)])
        for c in range(N_CHUNKS):
            b = c % 2
            if c + 1 < N_CHUNKS:
                in_dma[c + 1] = start_in(c + 1)
            in_dma[c].wait()
            if c >= 2:
                out_dma[c - 2].wait()
            idx_b = idx_v.at[b]
            val_b = val_v.at[b]

            def gather_row(r):
                for k in range(CHUNK_COLS // LANES):
                    iv = idx_b[r, pl.ds(k * LANES, LANES)]
                    val_b[r, pl.ds(k * LANES, LANES)] = plsc.load_gather(
                        tab_v, [iv]
                    )

            plsc.parallel_loop(0, TROWS, unroll=1)(gather_row)
            out_dma[c] = start_out(c)
        out_dma[N_CHUNKS - 2].wait()
        out_dma[N_CHUNKS - 1].wait()

    return sc_gather


_sc_gather_1000 = _make_sc_gather(1000)


def kernel(index, names_table):
    out_t = _sc_gather_1000(names_table, index.T)
    return out_t.T
